# speedup vs baseline: 5.3013x; 5.3013x over previous
import functools

import jax
import jax.numpy as jnp
from jax.experimental import pallas as pl


def _body(x_ref, w_ref, b_ref, gw_ref, gb_ref, o_ref, *, E):
    x = x_ref[...]
    glog = jnp.dot(x, gw_ref[...], preferred_element_type=jnp.float32) + gb_ref[...]
    eidx = jnp.argmax(glog, axis=-1)
    acc = jnp.zeros(o_ref.shape, jnp.float32)
    for e in range(E):
        ye = jnp.dot(x, w_ref[e], preferred_element_type=jnp.float32) + b_ref[e]
        acc = jnp.where((eidx == e)[:, None], ye, acc)
    o_ref[...] = jax.nn.softmax(acc, axis=-1)


def kernel(inputs, expert_W, expert_b, gate_W, gate_b):
    N, D = inputs.shape
    E, _, C = expert_W.shape
    BM = 256
    grid = (N // BM,)
    return pl.pallas_call(
        functools.partial(_body, E=E),
        grid=grid,
        in_specs=[
            pl.BlockSpec((BM, D), lambda i: (i, 0)),
            pl.BlockSpec((E, D, C), lambda i: (0, 0, 0)),
            pl.BlockSpec((E, C), lambda i: (0, 0)),
            pl.BlockSpec((D, E), lambda i: (0, 0)),
            pl.BlockSpec((1, E), lambda i: (0, 0)),
        ],
        out_specs=pl.BlockSpec((BM, C), lambda i: (i, 0)),
        out_shape=jax.ShapeDtypeStruct((N, C), jnp.float32),
    )(inputs, expert_W, expert_b, gate_W, gate_b.reshape(1, E))
